# baseline (speedup 1.0000x reference)
import functools

import jax
import jax.numpy as jnp
from jax import lax
from jax.experimental import pallas as pl
from jax.experimental.pallas import tpu as pltpu
from jax.experimental.pallas import tpu_sc as plsc

N = 10000
E = 320000
D = 128

NC = 2
NS = 16
NW = NC * NS
EPW = E // NW
CHUNK = 125
NCHUNK = EPW // CHUNK
RSTEP = 624
SPAN = 640
ZROWS = 128


def _sc_segsum(hb, src_r, dst_r, with_deg):
    mesh = plsc.VectorSubcoreMesh(core_axis_name="c", subcore_axis_name="s")

    @functools.partial(
        pl.kernel,
        mesh=mesh,
        compiler_params=pltpu.CompilerParams(use_tc_tiling_on_sc=False),
        out_type=(
            [jax.ShapeDtypeStruct((NC, N, D), jnp.bfloat16)]
            + ([jax.ShapeDtypeStruct((NC, N, 16), jnp.float32)]
               if with_deg else [])
        ),
        scratch_types=[
            pltpu.VMEM((NCHUNK, CHUNK), jnp.int32),
            pltpu.VMEM((NCHUNK, CHUNK), jnp.int32),
            [pltpu.VMEM((CHUNK, D), jnp.bfloat16)] * 6,
            pltpu.VMEM((CHUNK, 16), jnp.float32),
            pltpu.VMEM((ZROWS, D), jnp.bfloat16),
            pltpu.VMEM((ZROWS, 16), jnp.float32),
            pltpu.VMEM_SHARED((N, D), jnp.bfloat16),
            pltpu.VMEM_SHARED((N, 16), jnp.float32),
            [pltpu.SemaphoreType.DMA] * 6,
            [pltpu.SemaphoreType.DMA] * 6,
        ],
    )
    def seg_kernel(h_hbm, src_hbm, dst_hbm, spart_hbm, *rest):
        if with_deg:
            (degpart_hbm, src_v, dst_v, rows, ones_v, zrow_v, zdeg_v,
             s_acc, d_acc, sems, ssems) = rest
        else:
            (src_v, dst_v, rows, ones_v, zrow_v, zdeg_v,
             s_acc, d_acc, sems, ssems) = rest
        c = lax.axis_index("c")
        s = lax.axis_index("s")
        w = c * NS + s

        zero32b = jnp.zeros((32,), jnp.bfloat16)
        zero16 = jnp.zeros((16,), jnp.float32)
        one16 = jnp.ones((16,), jnp.float32)

        def fill_zrow(i, _):
            r = i // (D // 32)
            q = i % (D // 32)
            zrow_v[r, pl.ds(q * 32, 32)] = zero32b
            return 0
        lax.fori_loop(0, ZROWS * (D // 32), fill_zrow, 0)

        def fill_zdeg(i, _):
            zdeg_v[i, pl.ds(0, 16)] = zero16
            return 0
        lax.fori_loop(0, ZROWS, fill_zdeg, 0)

        def fill_ones(i, _):
            ones_v[i, pl.ds(0, 16)] = one16
            return 0
        lax.fori_loop(0, CHUNK, fill_ones, 0)

        pltpu.sync_copy(src_hbm.at[w], src_v)
        pltpu.sync_copy(dst_hbm.at[w], dst_v)

        base = s * RSTEP
        for k in range(SPAN // ZROWS):
            pltpu.sync_copy(zrow_v, s_acc.at[pl.ds(base + k * ZROWS, ZROWS)])
            if with_deg:
                pltpu.sync_copy(zdeg_v,
                                d_acc.at[pl.ds(base + k * ZROWS, ZROWS)])
        plsc.subcore_barrier()

        for q in range(3):
            pltpu.async_copy(h_hbm.at[src_v.at[q]], rows[q], sems[q])
        for j in range(NCHUNK):
            q = j % 6
            pltpu.make_async_copy(h_hbm.at[src_v.at[j]], rows[q],
                                  sems[q]).wait()
            pltpu.async_copy(rows[q], s_acc.at[dst_v.at[j]], ssems[q],
                             add=True)
            if with_deg:
                pltpu.sync_copy(ones_v, d_acc.at[dst_v.at[j]], add=True)
            nj = j + 3
            if nj < NCHUNK:
                nq = nj % 6
                if nj - 6 >= 0:
                    pltpu.make_async_copy(rows[nq],
                                          s_acc.at[dst_v.at[nj - 6]],
                                          ssems[nq]).wait()
                pltpu.async_copy(h_hbm.at[src_v.at[nj]], rows[nq], sems[nq])
        for j in range(NCHUNK - 6, NCHUNK):
            pltpu.make_async_copy(rows[j % 6], s_acc.at[dst_v.at[j]],
                                  ssems[j % 6]).wait()

        plsc.subcore_barrier()

        pltpu.sync_copy(s_acc.at[pl.ds(base, SPAN)],
                        spart_hbm.at[c, pl.ds(base, SPAN)])
        if with_deg:
            pltpu.sync_copy(d_acc.at[pl.ds(base, SPAN)],
                            degpart_hbm.at[c, pl.ds(base, SPAN)])

    return seg_kernel(hb, src_r, dst_r)


def _tc_layer(h, s_part, deg_part, w_self, w_neigh, b, bf16_out):
    blk = 400
    grid = (N // blk,)

    def body(h_ref, sp_ref, dg_ref, ws_ref, wn_ref, b_ref, *o_refs):
        deg = dg_ref[0, :, 0] + dg_ref[1, :, 0]
        r = 1.0 / jnp.maximum(deg, 1.0)
        ssum = (sp_ref[0].astype(jnp.float32)
                + sp_ref[1].astype(jnp.float32))
        hn = ssum * r[:, None]
        acc = jnp.dot(h_ref[...], ws_ref[...],
                      preferred_element_type=jnp.float32)
        acc += jnp.dot(hn, wn_ref[...],
                       preferred_element_type=jnp.float32)
        out = jnp.maximum(acc + b_ref[...], 0.0)
        o_refs[0][...] = out
        if bf16_out:
            o_refs[1][...] = out.astype(jnp.bfloat16)

    out_shape = [jax.ShapeDtypeStruct((N, D), jnp.float32)]
    out_specs = [pl.BlockSpec((blk, D), lambda i: (i, 0))]
    if bf16_out:
        out_shape.append(jax.ShapeDtypeStruct((N, D), jnp.bfloat16))
        out_specs.append(pl.BlockSpec((blk, D), lambda i: (i, 0)))

    return pl.pallas_call(
        body,
        grid=grid,
        in_specs=[
            pl.BlockSpec((blk, D), lambda i: (i, 0)),
            pl.BlockSpec((NC, blk, D), lambda i: (0, i, 0)),
            pl.BlockSpec((NC, blk, 16), lambda i: (0, i, 0)),
            pl.BlockSpec((D, D), lambda i: (0, 0)),
            pl.BlockSpec((D, D), lambda i: (0, 0)),
            pl.BlockSpec((1, D), lambda i: (0, 0)),
        ],
        out_specs=out_specs,
        out_shape=out_shape,
    )(h, s_part, deg_part, w_self, w_neigh, b)


def kernel(x, edge_index, W1_self, W1_neigh, b1, W2_self, W2_neigh, b2):
    src_r = edge_index[0].reshape(NW, NCHUNK, CHUNK)
    dst_r = edge_index[1].reshape(NW, NCHUNK, CHUNK)
    b1r = b1.reshape(1, D)
    b2r = b2.reshape(1, D)
    xb = x.astype(jnp.bfloat16)

    s1, dg1 = _sc_segsum(xb, src_r, dst_r, True)
    h1, h1b = _tc_layer(x, s1, dg1, W1_self, W1_neigh, b1r, True)
    (s2,) = _sc_segsum(h1b, src_r, dst_r, False)
    (out,) = _tc_layer(h1, s2, dg1, W2_self, W2_neigh, b2r, False)
    return out


# speedup vs baseline: 1.0283x; 1.0283x over previous
import functools

import jax
import jax.numpy as jnp
from jax import lax
from jax.experimental import pallas as pl
from jax.experimental.pallas import tpu as pltpu
from jax.experimental.pallas import tpu_sc as plsc

N = 10000
E = 320000
D = 128

NC = 2
NS = 16
NW = NC * NS
EPW = E // NW
CHUNK = 250
NCHUNK = EPW // CHUNK
RSTEP = 624
SPAN = 640
ZROWS = 64


def _sc_segsum(hb, src_r, dst_r, with_deg):
    mesh = plsc.VectorSubcoreMesh(core_axis_name="c", subcore_axis_name="s")

    @functools.partial(
        pl.kernel,
        mesh=mesh,
        compiler_params=pltpu.CompilerParams(use_tc_tiling_on_sc=False),
        out_type=(
            [jax.ShapeDtypeStruct((NC, N, D), jnp.bfloat16)]
            + ([jax.ShapeDtypeStruct((NC, N, 16), jnp.float32)]
               if with_deg else [])
        ),
        scratch_types=[
            pltpu.VMEM((NCHUNK, CHUNK), jnp.int32),
            pltpu.VMEM((NCHUNK, CHUNK), jnp.int32),
            [pltpu.VMEM((CHUNK, D), jnp.bfloat16)] * 3,
            pltpu.VMEM((CHUNK, 16), jnp.float32),
            pltpu.VMEM((ZROWS, D), jnp.bfloat16),
            pltpu.VMEM((ZROWS, 16), jnp.float32),
            pltpu.VMEM_SHARED((N, D), jnp.bfloat16),
            pltpu.VMEM_SHARED((N, 16), jnp.float32),
            [pltpu.SemaphoreType.DMA] * 3,
        ],
    )
    def seg_kernel(h_hbm, src_hbm, dst_hbm, spart_hbm, *rest):
        if with_deg:
            (degpart_hbm, src_v, dst_v, rows, ones_v, zrow_v, zdeg_v,
             s_acc, d_acc, sems) = rest
        else:
            (src_v, dst_v, rows, ones_v, zrow_v, zdeg_v,
             s_acc, d_acc, sems) = rest
        c = lax.axis_index("c")
        s = lax.axis_index("s")
        w = c * NS + s

        zero32b = jnp.zeros((32,), jnp.bfloat16)
        zero16 = jnp.zeros((16,), jnp.float32)
        one16 = jnp.ones((16,), jnp.float32)

        def fill_zrow(i, _):
            r = i // (D // 32)
            q = i % (D // 32)
            zrow_v[r, pl.ds(q * 32, 32)] = zero32b
            return 0
        lax.fori_loop(0, ZROWS * (D // 32), fill_zrow, 0)

        def fill_zdeg(i, _):
            zdeg_v[i, pl.ds(0, 16)] = zero16
            return 0
        lax.fori_loop(0, ZROWS, fill_zdeg, 0)

        def fill_ones(i, _):
            ones_v[i, pl.ds(0, 16)] = one16
            return 0
        lax.fori_loop(0, CHUNK, fill_ones, 0)

        pltpu.sync_copy(src_hbm.at[w], src_v)
        pltpu.sync_copy(dst_hbm.at[w], dst_v)

        base = s * RSTEP
        for k in range(SPAN // ZROWS):
            pltpu.sync_copy(zrow_v, s_acc.at[pl.ds(base + k * ZROWS, ZROWS)])
            if with_deg:
                pltpu.sync_copy(zdeg_v,
                                d_acc.at[pl.ds(base + k * ZROWS, ZROWS)])
        plsc.subcore_barrier()

        def consume(j, q):
            pltpu.make_async_copy(h_hbm.at[src_v.at[j]], rows[q],
                                  sems[q]).wait()
            pltpu.sync_copy(rows[q], s_acc.at[dst_v.at[j]], add=True)
            if with_deg:
                pltpu.sync_copy(ones_v, d_acc.at[dst_v.at[j]], add=True)

        for q in range(3):
            pltpu.async_copy(h_hbm.at[src_v.at[q]], rows[q], sems[q])
        for j in range(NCHUNK):
            consume(j, j % 3)
            if j + 3 < NCHUNK:
                pltpu.async_copy(h_hbm.at[src_v.at[j + 3]],
                                 rows[j % 3], sems[j % 3])

        plsc.subcore_barrier()

        pltpu.sync_copy(s_acc.at[pl.ds(base, SPAN)],
                        spart_hbm.at[c, pl.ds(base, SPAN)])
        if with_deg:
            pltpu.sync_copy(d_acc.at[pl.ds(base, SPAN)],
                            degpart_hbm.at[c, pl.ds(base, SPAN)])

    return seg_kernel(hb, src_r, dst_r)


def _tc_layer(h, s_part, deg_part, w_self, w_neigh, b, bf16_out):
    blk = 400
    grid = (N // blk,)

    def body(h_ref, sp_ref, dg_ref, ws_ref, wn_ref, b_ref, *o_refs):
        deg = dg_ref[0, :, 0] + dg_ref[1, :, 0]
        r = 1.0 / jnp.maximum(deg, 1.0)
        ssum = (sp_ref[0].astype(jnp.float32)
                + sp_ref[1].astype(jnp.float32))
        hn = ssum * r[:, None]
        acc = jnp.dot(h_ref[...], ws_ref[...],
                      preferred_element_type=jnp.float32)
        acc += jnp.dot(hn, wn_ref[...],
                       preferred_element_type=jnp.float32)
        out = jnp.maximum(acc + b_ref[...], 0.0)
        o_refs[0][...] = out
        if bf16_out:
            o_refs[1][...] = out.astype(jnp.bfloat16)

    out_shape = [jax.ShapeDtypeStruct((N, D), jnp.float32)]
    out_specs = [pl.BlockSpec((blk, D), lambda i: (i, 0))]
    if bf16_out:
        out_shape.append(jax.ShapeDtypeStruct((N, D), jnp.bfloat16))
        out_specs.append(pl.BlockSpec((blk, D), lambda i: (i, 0)))

    return pl.pallas_call(
        body,
        grid=grid,
        in_specs=[
            pl.BlockSpec((blk, D), lambda i: (i, 0)),
            pl.BlockSpec((NC, blk, D), lambda i: (0, i, 0)),
            pl.BlockSpec((NC, blk, 16), lambda i: (0, i, 0)),
            pl.BlockSpec((D, D), lambda i: (0, 0)),
            pl.BlockSpec((D, D), lambda i: (0, 0)),
            pl.BlockSpec((1, D), lambda i: (0, 0)),
        ],
        out_specs=out_specs,
        out_shape=out_shape,
    )(h, s_part, deg_part, w_self, w_neigh, b)


def kernel(x, edge_index, W1_self, W1_neigh, b1, W2_self, W2_neigh, b2):
    src_r = edge_index[0].reshape(NW, NCHUNK, CHUNK)
    dst_r = edge_index[1].reshape(NW, NCHUNK, CHUNK)
    b1r = b1.reshape(1, D)
    b2r = b2.reshape(1, D)
    xb = x.astype(jnp.bfloat16)

    s1, dg1 = _sc_segsum(xb, src_r, dst_r, True)
    h1, h1b = _tc_layer(x, s1, dg1, W1_self, W1_neigh, b1r, True)
    (s2,) = _sc_segsum(h1b, src_r, dst_r, False)
    (out,) = _tc_layer(h1, s2, dg1, W2_self, W2_neigh, b2r, False)
    return out


# speedup vs baseline: 1.0566x; 1.0275x over previous
import functools

import jax
import jax.numpy as jnp
from jax import lax
from jax.experimental import pallas as pl
from jax.experimental.pallas import tpu as pltpu
from jax.experimental.pallas import tpu_sc as plsc

N = 10000
E = 320000
D = 128

NC = 2
NS = 16
NW = NC * NS
EPW = E // NW
CHUNK = 200
NCHUNK = EPW // CHUNK
RSTEP = 624
SPAN = 640
ZROWS = 128


def _sc_segsum(hb, src_r, dst_r, with_deg):
    mesh = plsc.VectorSubcoreMesh(core_axis_name="c", subcore_axis_name="s")

    @functools.partial(
        pl.kernel,
        mesh=mesh,
        compiler_params=pltpu.CompilerParams(use_tc_tiling_on_sc=False),
        out_type=(
            [jax.ShapeDtypeStruct((NC, N, D), jnp.bfloat16)]
            + ([jax.ShapeDtypeStruct((NC, N, 16), jnp.float32)]
               if with_deg else [])
        ),
        scratch_types=[
            pltpu.VMEM((NCHUNK, CHUNK), jnp.int32),
            pltpu.VMEM((NCHUNK, CHUNK), jnp.int32),
            [pltpu.VMEM((CHUNK, D), jnp.bfloat16)] * 3,
            pltpu.VMEM((CHUNK, 16), jnp.float32),
            pltpu.VMEM((ZROWS, D), jnp.bfloat16),
            pltpu.VMEM((ZROWS, 16), jnp.float32),
            pltpu.VMEM_SHARED((N, D), jnp.bfloat16),
            pltpu.VMEM_SHARED((N, 16), jnp.float32),
            [pltpu.SemaphoreType.DMA] * 3,
        ],
    )
    def seg_kernel(h_hbm, src_hbm, dst_hbm, spart_hbm, *rest):
        if with_deg:
            (degpart_hbm, src_v, dst_v, rows, ones_v, zrow_v, zdeg_v,
             s_acc, d_acc, sems) = rest
        else:
            (src_v, dst_v, rows, ones_v, zrow_v, zdeg_v,
             s_acc, d_acc, sems) = rest
        c = lax.axis_index("c")
        s = lax.axis_index("s")
        w = c * NS + s

        zero32b = jnp.zeros((32,), jnp.bfloat16)
        zero16 = jnp.zeros((16,), jnp.float32)
        one16 = jnp.ones((16,), jnp.float32)

        def fill_zrow(i, _):
            r = i // (D // 32)
            q = i % (D // 32)
            zrow_v[r, pl.ds(q * 32, 32)] = zero32b
            return 0
        lax.fori_loop(0, ZROWS * (D // 32), fill_zrow, 0)

        def fill_zdeg(i, _):
            zdeg_v[i, pl.ds(0, 16)] = zero16
            return 0
        lax.fori_loop(0, ZROWS, fill_zdeg, 0)

        def fill_ones(i, _):
            ones_v[i, pl.ds(0, 16)] = one16
            return 0
        lax.fori_loop(0, CHUNK, fill_ones, 0)

        pltpu.sync_copy(src_hbm.at[w], src_v)
        pltpu.sync_copy(dst_hbm.at[w], dst_v)

        base = s * RSTEP
        for k in range(SPAN // ZROWS):
            pltpu.sync_copy(zrow_v, s_acc.at[pl.ds(base + k * ZROWS, ZROWS)])
            if with_deg:
                pltpu.sync_copy(zdeg_v,
                                d_acc.at[pl.ds(base + k * ZROWS, ZROWS)])
        plsc.subcore_barrier()

        def consume(j, q):
            pltpu.make_async_copy(h_hbm.at[src_v.at[j]], rows[q],
                                  sems[q]).wait()
            pltpu.sync_copy(rows[q], s_acc.at[dst_v.at[j]], add=True)
            if with_deg:
                pltpu.sync_copy(ones_v, d_acc.at[dst_v.at[j]], add=True)

        for q in range(3):
            pltpu.async_copy(h_hbm.at[src_v.at[q]], rows[q], sems[q])
        for j in range(NCHUNK):
            consume(j, j % 3)
            if j + 3 < NCHUNK:
                pltpu.async_copy(h_hbm.at[src_v.at[j + 3]],
                                 rows[j % 3], sems[j % 3])

        plsc.subcore_barrier()

        pltpu.sync_copy(s_acc.at[pl.ds(base, SPAN)],
                        spart_hbm.at[c, pl.ds(base, SPAN)])
        if with_deg:
            pltpu.sync_copy(d_acc.at[pl.ds(base, SPAN)],
                            degpart_hbm.at[c, pl.ds(base, SPAN)])

    return seg_kernel(hb, src_r, dst_r)


def _tc_layer(h, s_part, deg_part, w_self, w_neigh, b, bf16_out):
    blk = 400
    grid = (N // blk,)

    def body(h_ref, sp_ref, dg_ref, ws_ref, wn_ref, b_ref, *o_refs):
        deg = dg_ref[0, :, 0] + dg_ref[1, :, 0]
        r = 1.0 / jnp.maximum(deg, 1.0)
        ssum = (sp_ref[0].astype(jnp.float32)
                + sp_ref[1].astype(jnp.float32))
        hn = ssum * r[:, None]
        acc = jnp.dot(h_ref[...], ws_ref[...],
                      preferred_element_type=jnp.float32)
        acc += jnp.dot(hn, wn_ref[...],
                       preferred_element_type=jnp.float32)
        out = jnp.maximum(acc + b_ref[...], 0.0)
        o_refs[0][...] = out
        if bf16_out:
            o_refs[1][...] = out.astype(jnp.bfloat16)

    out_shape = [jax.ShapeDtypeStruct((N, D), jnp.float32)]
    out_specs = [pl.BlockSpec((blk, D), lambda i: (i, 0))]
    if bf16_out:
        out_shape.append(jax.ShapeDtypeStruct((N, D), jnp.bfloat16))
        out_specs.append(pl.BlockSpec((blk, D), lambda i: (i, 0)))

    return pl.pallas_call(
        body,
        grid=grid,
        in_specs=[
            pl.BlockSpec((blk, D), lambda i: (i, 0)),
            pl.BlockSpec((NC, blk, D), lambda i: (0, i, 0)),
            pl.BlockSpec((NC, blk, 16), lambda i: (0, i, 0)),
            pl.BlockSpec((D, D), lambda i: (0, 0)),
            pl.BlockSpec((D, D), lambda i: (0, 0)),
            pl.BlockSpec((1, D), lambda i: (0, 0)),
        ],
        out_specs=out_specs,
        out_shape=out_shape,
    )(h, s_part, deg_part, w_self, w_neigh, b)


def kernel(x, edge_index, W1_self, W1_neigh, b1, W2_self, W2_neigh, b2):
    src_r = edge_index[0].reshape(NW, NCHUNK, CHUNK)
    dst_r = edge_index[1].reshape(NW, NCHUNK, CHUNK)
    b1r = b1.reshape(1, D)
    b2r = b2.reshape(1, D)
    xb = x.astype(jnp.bfloat16)

    s1, dg1 = _sc_segsum(xb, src_r, dst_r, True)
    h1, h1b = _tc_layer(x, s1, dg1, W1_self, W1_neigh, b1r, True)
    (s2,) = _sc_segsum(h1b, src_r, dst_r, False)
    (out,) = _tc_layer(h1, s2, dg1, W2_self, W2_neigh, b2r, False)
    return out


# speedup vs baseline: 1.0613x; 1.0044x over previous
import functools

import jax
import jax.numpy as jnp
from jax import lax
from jax.experimental import pallas as pl
from jax.experimental.pallas import tpu as pltpu
from jax.experimental.pallas import tpu_sc as plsc

N = 10000
E = 320000
D = 128

NC = 2
NS = 16
NW = NC * NS
EPW = E // NW
CHUNK = 200
NCHUNK = EPW // CHUNK
RSTEP = 624
SPAN = 640
ZROWS = 128


def _sc_segsum(hb, src_r, dst_r, with_deg):
    mesh = plsc.VectorSubcoreMesh(core_axis_name="c", subcore_axis_name="s")

    @functools.partial(
        pl.kernel,
        mesh=mesh,
        compiler_params=pltpu.CompilerParams(use_tc_tiling_on_sc=False),
        out_type=(
            [jax.ShapeDtypeStruct((NC, N, D), jnp.bfloat16)]
            + ([jax.ShapeDtypeStruct((NC, N, 16), jnp.float32)]
               if with_deg else [])
        ),
        scratch_types=[
            pltpu.VMEM((NCHUNK, CHUNK), jnp.int32),
            pltpu.VMEM((NCHUNK, CHUNK), jnp.int32),
            [pltpu.VMEM((CHUNK, D), jnp.bfloat16)] * 3,
            pltpu.VMEM((CHUNK, 16), jnp.float32),
            pltpu.VMEM((ZROWS, D), jnp.bfloat16),
            pltpu.VMEM((ZROWS, 16), jnp.float32),
            pltpu.VMEM_SHARED((N, D), jnp.bfloat16),
            pltpu.VMEM_SHARED((N, 16), jnp.float32),
            [pltpu.SemaphoreType.DMA] * 3,
        ],
    )
    def seg_kernel(h_hbm, src_hbm, dst_hbm, spart_hbm, *rest):
        if with_deg:
            (degpart_hbm, src_v, dst_v, rows, ones_v, zrow_v, zdeg_v,
             s_acc, d_acc, sems) = rest
        else:
            (src_v, dst_v, rows, ones_v, zrow_v, zdeg_v,
             s_acc, d_acc, sems) = rest
        c = lax.axis_index("c")
        s = lax.axis_index("s")
        w = c * NS + s

        zero32b = jnp.zeros((32,), jnp.bfloat16)
        zero16 = jnp.zeros((16,), jnp.float32)
        one16 = jnp.ones((16,), jnp.float32)

        def fill_zrow(i, _):
            r = i // (D // 32)
            q = i % (D // 32)
            zrow_v[r, pl.ds(q * 32, 32)] = zero32b
            return 0
        lax.fori_loop(0, ZROWS * (D // 32), fill_zrow, 0)

        def fill_zdeg(i, _):
            zdeg_v[i, pl.ds(0, 16)] = zero16
            return 0
        lax.fori_loop(0, ZROWS, fill_zdeg, 0)

        def fill_ones(i, _):
            ones_v[i, pl.ds(0, 16)] = one16
            return 0
        lax.fori_loop(0, CHUNK, fill_ones, 0)

        pltpu.sync_copy(src_hbm.at[w], src_v)
        pltpu.sync_copy(dst_hbm.at[w], dst_v)

        base = s * RSTEP
        for k in range(SPAN // ZROWS):
            pltpu.sync_copy(zrow_v, s_acc.at[pl.ds(base + k * ZROWS, ZROWS)])
            if with_deg:
                pltpu.sync_copy(zdeg_v,
                                d_acc.at[pl.ds(base + k * ZROWS, ZROWS)])
        plsc.subcore_barrier()

        def consume(j, q):
            pltpu.make_async_copy(h_hbm.at[src_v.at[j]], rows[q],
                                  sems[q]).wait()
            pltpu.sync_copy(rows[q], s_acc.at[dst_v.at[j]], add=True)
            if with_deg:
                pltpu.sync_copy(ones_v, d_acc.at[dst_v.at[j]], add=True)

        for q in range(3):
            pltpu.async_copy(h_hbm.at[src_v.at[q]], rows[q], sems[q])
        for j in range(NCHUNK):
            consume(j, j % 3)
            if j + 3 < NCHUNK:
                pltpu.async_copy(h_hbm.at[src_v.at[j + 3]],
                                 rows[j % 3], sems[j % 3])

        plsc.subcore_barrier()

        pltpu.sync_copy(s_acc.at[pl.ds(base, SPAN)],
                        spart_hbm.at[c, pl.ds(base, SPAN)])
        if with_deg:
            pltpu.sync_copy(d_acc.at[pl.ds(base, SPAN)],
                            degpart_hbm.at[c, pl.ds(base, SPAN)])

    return seg_kernel(hb, src_r, dst_r)


def _tc_layer(h, s_part, deg_part, w_self, w_neigh, b, out_dtype):
    blk = 400
    grid = (N // blk,)

    def body(h_ref, sp_ref, dg_ref, ws_ref, wn_ref, b_ref, o_ref):
        deg = dg_ref[0, :, 0] + dg_ref[1, :, 0]
        r = 1.0 / jnp.maximum(deg, 1.0)
        ssum = (sp_ref[0].astype(jnp.float32)
                + sp_ref[1].astype(jnp.float32))
        hn = ssum * r[:, None]
        acc = jnp.dot(h_ref[...], ws_ref[...],
                      preferred_element_type=jnp.float32)
        acc += jnp.dot(hn, wn_ref[...],
                       preferred_element_type=jnp.float32)
        out = jnp.maximum(acc + b_ref[...], 0.0)
        o_ref[...] = out.astype(out_dtype)

    return pl.pallas_call(
        body,
        grid=grid,
        in_specs=[
            pl.BlockSpec((blk, D), lambda i: (i, 0)),
            pl.BlockSpec((NC, blk, D), lambda i: (0, i, 0)),
            pl.BlockSpec((NC, blk, 16), lambda i: (0, i, 0)),
            pl.BlockSpec((D, D), lambda i: (0, 0)),
            pl.BlockSpec((D, D), lambda i: (0, 0)),
            pl.BlockSpec((1, D), lambda i: (0, 0)),
        ],
        out_specs=pl.BlockSpec((blk, D), lambda i: (i, 0)),
        out_shape=jax.ShapeDtypeStruct((N, D), out_dtype),
    )(h, s_part, deg_part, w_self, w_neigh, b)


def kernel(x, edge_index, W1_self, W1_neigh, b1, W2_self, W2_neigh, b2):
    src_r = edge_index[0].reshape(NW, NCHUNK, CHUNK)
    dst_r = edge_index[1].reshape(NW, NCHUNK, CHUNK)
    b1r = b1.reshape(1, D)
    b2r = b2.reshape(1, D)
    xb = x.astype(jnp.bfloat16)

    s1, dg1 = _sc_segsum(xb, src_r, dst_r, True)
    h1b = _tc_layer(x, s1, dg1, W1_self, W1_neigh, b1r, jnp.bfloat16)
    (s2,) = _sc_segsum(h1b, src_r, dst_r, False)
    out = _tc_layer(h1b, s2, dg1, W2_self, W2_neigh, b2r, jnp.float32)
    return out
